# baseline (speedup 1.0000x reference)
import functools

import jax
import jax.numpy as jnp
from jax import lax
from jax.experimental import pallas as pl
from jax.experimental.pallas import tpu as pltpu
from jax.experimental.pallas import tpu_sc as plsc

NTOKEN = 100000
D = 1024
B = 4
S = 2048
EPS = 1e-5
L = 16
NW = 32
S_PER_W = S // NW
CH = 16
NCH = S_PER_W // CH
NCOL = D // L


def _rsqrt(x):
    i = lax.bitcast_convert_type(x, jnp.int32)
    i = jnp.int32(0x5F3759DF) - lax.shift_right_logical(i, 1)
    y = lax.bitcast_convert_type(i, jnp.float32)
    for _ in range(3):
        y = y * (1.5 - 0.5 * x * y * y)
    return y


def _sc_body(ids_hbm, vals_hbm, table_hbm, pos_hbm, w_hbm, b_hbm,
             gamma_hbm, beta_hbm, out_hbm,
             idx_v, vals_v, w_v, b_v, gamma_v, beta_v,
             gene_buf, pos_buf, sem):
    nc = 2
    wid = lax.axis_index("s") * nc + lax.axis_index("c")
    s0 = wid * S_PER_W

    for b in range(B):
        pltpu.sync_copy(ids_hbm.at[pl.ds(b * S + s0, S_PER_W)],
                        idx_v.at[pl.ds(b * S_PER_W, S_PER_W)])
        pltpu.sync_copy(vals_hbm.at[pl.ds(b * S + s0, S_PER_W)],
                        vals_v.at[pl.ds(b * S_PER_W, S_PER_W)])
    pltpu.sync_copy(w_hbm, w_v)
    pltpu.sync_copy(b_hbm, b_v)
    pltpu.sync_copy(gamma_hbm, gamma_v)
    pltpu.sync_copy(beta_hbm, beta_v)

    lane = lax.iota(jnp.int32, L)
    zero = jnp.zeros((L,), jnp.float32)

    def row_body(r, carry):
        v16 = carry
        vr = jnp.sum(jnp.where(lane == r, v16, 0.0))

        def col1(j, acc2):
            acc, accsq = acc2
            off = j * L
            g = gene_buf[r, pl.ds(off, L)]
            p = pos_buf[r, pl.ds(off, L)]
            e = g + p + jnp.maximum(vr * w_v[pl.ds(off, L)]
                                    + b_v[pl.ds(off, L)], 0.0)
            gene_buf[r, pl.ds(off, L)] = e
            return acc + e, accsq + e * e

        acc, accsq = lax.fori_loop(0, NCOL, col1, (zero, zero))
        mean = jnp.sum(acc) * (1.0 / D)
        var = jnp.sum(accsq) * (1.0 / D) - mean * mean
        rstd = _rsqrt(var + EPS)

        def col2(j, c):
            off = j * L
            e = gene_buf[r, pl.ds(off, L)]
            gene_buf[r, pl.ds(off, L)] = ((e - mean) * rstd
                                          * gamma_v[pl.ds(off, L)]
                                          + beta_v[pl.ds(off, L)])
            return c

        lax.fori_loop(0, NCOL, col2, 0)
        return v16

    for sc in range(NCH):
        spos = s0 + sc * CH
        pltpu.sync_copy(pos_hbm.at[pl.ds(spos, CH), :], pos_buf)
        for b in range(B):
            ibase = b * S_PER_W + sc * CH
            pltpu.async_copy(
                table_hbm.at[idx_v.at[pl.ds(ibase, CH)]], gene_buf, sem
            ).wait()
            v16 = vals_v[pl.ds(ibase, L)]
            lax.fori_loop(0, CH, row_body, v16)
            pltpu.sync_copy(gene_buf,
                            out_hbm.at[pl.ds(b * S + spos, CH), :])


def kernel(gene_ids, values, gene_table, pos_table, W_val, b_val, gamma, beta):
    ids_flat = gene_ids.reshape(-1).astype(jnp.int32)
    vals_flat = values.reshape(-1).astype(jnp.float32)
    w_flat = W_val.reshape(-1)

    mesh = plsc.VectorSubcoreMesh(core_axis_name="c", subcore_axis_name="s")

    k = pl.kernel(
        _sc_body,
        jax.ShapeDtypeStruct((B * S, D), jnp.float32),
        mesh=mesh,
        compiler_params=pltpu.CompilerParams(needs_layout_passes=False),
        scratch_types=[
            pltpu.VMEM((B * S_PER_W,), jnp.int32),
            pltpu.VMEM((B * S_PER_W,), jnp.float32),
            pltpu.VMEM((D,), jnp.float32),
            pltpu.VMEM((D,), jnp.float32),
            pltpu.VMEM((D,), jnp.float32),
            pltpu.VMEM((D,), jnp.float32),
            pltpu.VMEM((CH, D), jnp.float32),
            pltpu.VMEM((CH, D), jnp.float32),
            pltpu.SemaphoreType.DMA,
        ],
    )
    out = k(ids_flat, vals_flat, gene_table, pos_table,
            w_flat, b_val, gamma, beta)
    return out.reshape(B, S, D)
